# Initial kernel scaffold; baseline (speedup 1.0000x reference)
import jax
import jax.numpy as jnp
from jax.experimental import pallas as pl


def kernel(X, adj, node_mask, W_in, b_in, Wg0, bg0, Wg1, bg1, Wg2, bg2, Wo1, bo1, Wo2, bo2):
    raise NotImplementedError("write your pallas kernel here")


# speedup vs baseline: 1.4114x; 1.4114x over previous
import jax
import jax.numpy as jnp
from jax import lax
from jax.experimental import pallas as pl


def _leaky(x):
    return jnp.where(x >= 0, x, 0.01 * x)


def _gcn_body(A_ref, X_ref, m_ref, Win_ref, bin_ref, Wg0_ref, bg0_ref,
              Wg1_ref, bg1_ref, Wg2_ref, bg2_ref, Wo1_ref, bo1_ref,
              Wo2_ref, bo2_ref, out_ref):
    A = A_ref[0]
    deg = jnp.sum(A, axis=0) + 1.0
    dinv = lax.rsqrt(deg)

    H = _leaky(jnp.dot(X_ref[0], Win_ref[...],
                       preferred_element_type=jnp.float32) + bin_ref[...])
    for W_ref, b_ref in ((Wg0_ref, bg0_ref), (Wg1_ref, bg1_ref),
                         (Wg2_ref, bg2_ref)):
        M = jnp.dot(H, W_ref[...], preferred_element_type=jnp.float32)
        Ms = M * dinv[:, None]
        Y = lax.dot_general(A, Ms, (((0,), (0,)), ((), ())),
                            preferred_element_type=jnp.float32) + Ms
        H = _leaky(Y * dinv[:, None] + b_ref[...])

    Xo = jnp.dot(_leaky(jnp.dot(H, Wo1_ref[...],
                                preferred_element_type=jnp.float32)
                        + bo1_ref[...]),
                 Wo2_ref[...], preferred_element_type=jnp.float32)
    Xo = Xo + bo2_ref[...]
    out_ref[0] = Xo * m_ref[0]


def _mask_e_body(adj_ref, mrow_ref, mcol_ref, out_ref):
    out_ref[0] = adj_ref[0] * mrow_ref[0] * mcol_ref[0]


def kernel(X, adj, node_mask, W_in, b_in, Wg0, bg0, Wg1, bg1, Wg2, bg2,
           Wo1, bo1, Wo2, bo2):
    bs, n, d_in = X.shape
    d_out = Wo2.shape[1]
    A3 = adj.reshape(bs, n, n)
    m_row = node_mask.reshape(bs, n, 1)
    m_col = node_mask.reshape(bs, 1, n)

    def v(b):
        return b.reshape(1, -1)

    full2 = lambda s: pl.BlockSpec(s, lambda i: (0, 0))
    X_out = pl.pallas_call(
        _gcn_body,
        grid=(bs,),
        in_specs=[
            pl.BlockSpec((1, n, n), lambda i: (i, 0, 0)),
            pl.BlockSpec((1, n, d_in), lambda i: (i, 0, 0)),
            pl.BlockSpec((1, n, 1), lambda i: (i, 0, 0)),
            full2(W_in.shape), full2((1, b_in.shape[0])),
            full2(Wg0.shape), full2((1, bg0.shape[0])),
            full2(Wg1.shape), full2((1, bg1.shape[0])),
            full2(Wg2.shape), full2((1, bg2.shape[0])),
            full2(Wo1.shape), full2((1, bo1.shape[0])),
            full2(Wo2.shape), full2((1, bo2.shape[0])),
        ],
        out_specs=pl.BlockSpec((1, n, d_out), lambda i: (i, 0, 0)),
        out_shape=jax.ShapeDtypeStruct((bs, n, d_out), jnp.float32),
    )(A3, X, m_row, W_in, v(b_in), Wg0, v(bg0), Wg1, v(bg1), Wg2, v(bg2),
      Wo1, v(bo1), Wo2, v(bo2))

    blk = 512
    E3 = pl.pallas_call(
        _mask_e_body,
        grid=(bs, n // blk),
        in_specs=[
            pl.BlockSpec((1, blk, n), lambda i, j: (i, j, 0)),
            pl.BlockSpec((1, blk, 1), lambda i, j: (i, j, 0)),
            pl.BlockSpec((1, 1, n), lambda i, j: (i, 0, 0)),
        ],
        out_specs=pl.BlockSpec((1, blk, n), lambda i, j: (i, j, 0)),
        out_shape=jax.ShapeDtypeStruct((bs, n, n), jnp.float32),
    )(A3, m_row, m_col)
    return X_out, E3.reshape(bs, n, n, 1)
